# baseline (speedup 1.0000x reference)
import functools

import jax
import jax.numpy as jnp
from jax import lax
from jax.experimental import pallas as pl
from jax.experimental.pallas import tpu as pltpu
from jax.experimental.pallas import tpu_sc as plsc

N_NODES = 100000
EMBED_DIM = 128
BATCH = 16384

L = 16
NUM_CORES = 2
NUM_SUBCORES = 16
NW = NUM_CORES * NUM_SUBCORES
B_PER_W = BATCH // NW
CHUNK = 128
CH = ((0, 128), (128, 128), (256, 128), (384, 96), (480, 32))
NBUF = 2
N_SEG = EMBED_DIM // L
TR_STRIDE = L + 1

_mesh = plsc.VectorSubcoreMesh(core_axis_name="c", subcore_axis_name="s")


@functools.partial(
    pl.kernel,
    mesh=_mesh,
    out_type=jax.ShapeDtypeStruct((BATCH,), jnp.float32),
    scratch_types=[
        pltpu.VMEM((B_PER_W,), jnp.int32),
        pltpu.VMEM((B_PER_W,), jnp.int32),
        [pltpu.VMEM((CHUNK, EMBED_DIM), jnp.float32)] * NBUF,
        [pltpu.VMEM((CHUNK, EMBED_DIM), jnp.float32)] * NBUF,
        pltpu.VMEM((EMBED_DIM,), jnp.float32),
        pltpu.VMEM((B_PER_W,), jnp.float32),
        pltpu.VMEM((L * TR_STRIDE,), jnp.float32),
        [pltpu.SemaphoreType.DMA] * NBUF,
        [pltpu.SemaphoreType.DMA] * NBUF,
        pltpu.SemaphoreType.DMA,
    ],
    compiler_params=pltpu.CompilerParams(
        needs_layout_passes=False,
        skip_device_barrier=True,
        disable_bounds_checks=True,
        disable_semaphore_checks=True,
    ),
)
def _distmult_sc(head_hbm, tail_hbm, table_hbm, rel_hbm, out_hbm,
                 hidx_v, tidx_v, h_bufs, t_bufs, r_v, o_v, tr_v,
                 sems_h, sems_t, sem_p):
    wid = lax.axis_index("s") * NUM_CORES + lax.axis_index("c")
    base = wid * B_PER_W

    cp_hi = pltpu.async_copy(head_hbm.at[pl.ds(base, B_PER_W)], hidx_v,
                             sems_h[0])
    cp_ti = pltpu.async_copy(tail_hbm.at[pl.ds(base, B_PER_W)], tidx_v,
                             sems_t[0])
    cp_r = pltpu.async_copy(rel_hbm, r_v, sem_p)
    cp_hi.wait()
    cp_ti.wait()

    tr_idx = lax.iota(jnp.int32, L) * TR_STRIDE

    def _issue(i):
        p = i % NBUF
        start, n = CH[i]
        cp_h = pltpu.async_copy(
            table_hbm.at[hidx_v.at[pl.ds(start, n)]],
            h_bufs[p].at[pl.ds(0, n)], sems_h[p])
        cp_t = pltpu.async_copy(
            table_hbm.at[tidx_v.at[pl.ds(start, n)]],
            t_bufs[p].at[pl.ds(0, n)], sems_t[p])
        return cp_h, cp_t

    pending = [_issue(i) for i in range(NBUF)]
    cp_r.wait()
    for c in range(len(CH)):
        p = c % NBUF
        start, n = CH[c]
        h_v, t_v = h_bufs[p], t_bufs[p]
        pending[p][0].wait()
        pending[p][1].wait()

        def _groups(g, rsegs, start=start, h_v=h_v, t_v=t_v):
            b0 = g * L
            for ub in range(0, L, 4):
                accs = [None] * 4
                for k in range(N_SEG):
                    for u in range(4):
                        p_ = (h_v[b0 + ub + u, pl.ds(k * L, L)]
                              * t_v[b0 + ub + u, pl.ds(k * L, L)]
                              * rsegs[k])
                        accs[u] = p_ if k == 0 else accs[u] + p_
                for u in range(4):
                    plsc.store_scatter(tr_v, [tr_idx + ub + u], accs[u])
            sv = tr_v[pl.ds(0, L)]
            for l in range(1, L):
                sv = sv + tr_v[pl.ds(l * TR_STRIDE, L)]
            o_v[pl.ds(start + g * L, L)] = sv
            return rsegs

        rsegs0 = tuple(r_v[pl.ds(k * L, L)] for k in range(N_SEG))
        lax.fori_loop(0, n // L, _groups, rsegs0)

        if c + NBUF < len(CH):
            pending[p] = _issue(c + NBUF)

    pltpu.sync_copy(o_v, out_hbm.at[pl.ds(base, B_PER_W)])


def kernel(head_indices, tail_indices, node_embedding, relation_vector):
    return _distmult_sc(head_indices, tail_indices, node_embedding,
                        relation_vector)


# speedup vs baseline: 1.0794x; 1.0794x over previous
import functools

import jax
import jax.numpy as jnp
from jax import lax
from jax.experimental import pallas as pl
from jax.experimental.pallas import tpu as pltpu
from jax.experimental.pallas import tpu_sc as plsc

N_NODES = 100000
EMBED_DIM = 128
BATCH = 16384

L = 16
NUM_CORES = 2
NUM_SUBCORES = 16
NW = NUM_CORES * NUM_SUBCORES
B_PER_W = BATCH // NW
CHUNK = 128
N_CHUNKS = B_PER_W // CHUNK
NBUF = 2
N_SEG = EMBED_DIM // L
TR_STRIDE = L + 1

_mesh = plsc.VectorSubcoreMesh(core_axis_name="c", subcore_axis_name="s")


@functools.partial(
    pl.kernel,
    mesh=_mesh,
    out_type=jax.ShapeDtypeStruct((BATCH,), jnp.float32),
    scratch_types=[
        pltpu.VMEM((B_PER_W,), jnp.int32),
        pltpu.VMEM((B_PER_W,), jnp.int32),
        [pltpu.VMEM((CHUNK, EMBED_DIM), jnp.float32)] * NBUF,
        [pltpu.VMEM((CHUNK, EMBED_DIM), jnp.float32)] * NBUF,
        pltpu.VMEM((EMBED_DIM,), jnp.float32),
        pltpu.VMEM((B_PER_W,), jnp.float32),
        pltpu.VMEM((L * TR_STRIDE,), jnp.float32),
        [pltpu.SemaphoreType.DMA] * NBUF,
        [pltpu.SemaphoreType.DMA] * NBUF,
        pltpu.SemaphoreType.DMA,
    ],
    compiler_params=pltpu.CompilerParams(
        needs_layout_passes=False,
        skip_device_barrier=True,
        disable_bounds_checks=True,
        disable_semaphore_checks=True,
    ),
)
def _distmult_sc(head_hbm, tail_hbm, table_hbm, rel_hbm, out_hbm,
                 hidx_v, tidx_v, h_bufs, t_bufs, r_v, o_v, tr_v,
                 sems_h, sems_t, sem_p):
    wid = lax.axis_index("s") * NUM_CORES + lax.axis_index("c")
    base = wid * B_PER_W

    cp_hi = pltpu.async_copy(head_hbm.at[pl.ds(base, B_PER_W)], hidx_v,
                             sems_h[0])
    cp_ti = pltpu.async_copy(tail_hbm.at[pl.ds(base, B_PER_W)], tidx_v,
                             sems_t[0])
    cp_r = pltpu.async_copy(rel_hbm, r_v, sem_p)
    cp_hi.wait()
    cp_ti.wait()

    tr_idx = lax.iota(jnp.int32, L) * TR_STRIDE

    def _issue(c, p):
        pltpu.async_copy(
            table_hbm.at[hidx_v.at[pl.ds(c * CHUNK, CHUNK)]],
            h_bufs[p], sems_h[p])
        pltpu.async_copy(
            table_hbm.at[tidx_v.at[pl.ds(c * CHUNK, CHUNK)]],
            t_bufs[p], sems_t[p])

    for i in range(NBUF):
        _issue(i, i)
    cp_r.wait()

    rsegs0 = tuple(r_v[pl.ds(k * L, L)] for k in range(N_SEG))

    def _pair(cp, rsegs_outer):
        rsegs_cur = rsegs_outer
        for b in range(NBUF):
            c = cp * NBUF + b
            h_v, t_v = h_bufs[b], t_bufs[b]
            pltpu.make_async_copy(
                table_hbm.at[pl.ds(0, CHUNK)], h_v, sems_h[b]).wait()
            pltpu.make_async_copy(
                table_hbm.at[pl.ds(0, CHUNK)], t_v, sems_t[b]).wait()

            def _groups(g, rsegs, c=c, h_v=h_v, t_v=t_v):
                b0 = g * L
                for ub in range(0, L, 4):
                    accs = [None] * 4
                    for k in range(N_SEG):
                        for u in range(4):
                            p_ = (h_v[b0 + ub + u, pl.ds(k * L, L)]
                                  * t_v[b0 + ub + u, pl.ds(k * L, L)]
                                  * rsegs[k])
                            accs[u] = p_ if k == 0 else accs[u] + p_
                    for u in range(4):
                        plsc.store_scatter(tr_v, [tr_idx + ub + u], accs[u])
                sv = tr_v[pl.ds(0, L)]
                for l in range(1, L):
                    sv = sv + tr_v[pl.ds(l * TR_STRIDE, L)]
                o_v[pl.ds(c * CHUNK + g * L, L)] = sv
                return rsegs

            rsegs_cur = lax.fori_loop(0, CHUNK // L, _groups, rsegs_cur)

            @pl.when(c + NBUF < N_CHUNKS)
            def _():
                _issue(c + NBUF, b)
        return rsegs_cur

    lax.fori_loop(0, N_CHUNKS // NBUF, _pair, rsegs0)

    pltpu.sync_copy(o_v, out_hbm.at[pl.ds(base, B_PER_W)])


def kernel(head_indices, tail_indices, node_embedding, relation_vector):
    return _distmult_sc(head_indices, tail_indices, node_embedding,
                        relation_vector)


# speedup vs baseline: 1.0989x; 1.0181x over previous
import functools

import jax
import jax.numpy as jnp
from jax import lax
from jax.experimental import pallas as pl
from jax.experimental.pallas import tpu as pltpu
from jax.experimental.pallas import tpu_sc as plsc

N_NODES = 100000
EMBED_DIM = 128
BATCH = 16384

L = 16
NUM_CORES = 2
NUM_SUBCORES = 16
NW = NUM_CORES * NUM_SUBCORES
B_PER_W = BATCH // NW
CHUNK = 128
N_CHUNKS = B_PER_W // CHUNK
NBUF = 2
N_SEG = EMBED_DIM // L
TR_STRIDE = L + 1

_mesh = plsc.VectorSubcoreMesh(core_axis_name="c", subcore_axis_name="s")


@functools.partial(
    pl.kernel,
    mesh=_mesh,
    out_type=jax.ShapeDtypeStruct((BATCH,), jnp.float32),
    scratch_types=[
        pltpu.VMEM((B_PER_W,), jnp.int32),
        pltpu.VMEM((B_PER_W,), jnp.int32),
        pltpu.VMEM((NBUF * CHUNK, EMBED_DIM), jnp.float32),
        pltpu.VMEM((NBUF * CHUNK, EMBED_DIM), jnp.float32),
        pltpu.VMEM((EMBED_DIM,), jnp.float32),
        pltpu.VMEM((B_PER_W,), jnp.float32),
        pltpu.VMEM((L * TR_STRIDE,), jnp.float32),
        [pltpu.SemaphoreType.DMA] * NBUF,
        [pltpu.SemaphoreType.DMA] * NBUF,
        pltpu.SemaphoreType.DMA,
    ],
    compiler_params=pltpu.CompilerParams(
        needs_layout_passes=False,
        skip_device_barrier=True,
        disable_bounds_checks=True,
        disable_semaphore_checks=True,
    ),
)
def _distmult_sc(head_hbm, tail_hbm, table_hbm, rel_hbm, out_hbm,
                 hidx_v, tidx_v, h_bufs, t_bufs, r_v, o_v, tr_v,
                 sems_h, sems_t, sem_p):
    wid = lax.axis_index("s") * NUM_CORES + lax.axis_index("c")
    base = wid * B_PER_W

    cp_hi = pltpu.async_copy(head_hbm.at[pl.ds(base, B_PER_W)], hidx_v,
                             sems_h[0])
    cp_ti = pltpu.async_copy(tail_hbm.at[pl.ds(base, B_PER_W)], tidx_v,
                             sems_t[0])
    cp_r = pltpu.async_copy(rel_hbm, r_v, sem_p)
    cp_hi.wait()
    cp_ti.wait()

    tr_idx = lax.iota(jnp.int32, L) * TR_STRIDE

    def _issue(c, p):
        pltpu.async_copy(
            table_hbm.at[hidx_v.at[pl.ds(c * CHUNK, CHUNK)]],
            h_bufs.at[pl.ds(p * CHUNK, CHUNK)], sems_h[p])
        pltpu.async_copy(
            table_hbm.at[tidx_v.at[pl.ds(c * CHUNK, CHUNK)]],
            t_bufs.at[pl.ds(p * CHUNK, CHUNK)], sems_t[p])

    for i in range(NBUF):
        _issue(i, i)
    cp_r.wait()

    rsegs0 = tuple(r_v[pl.ds(k * L, L)] for k in range(N_SEG))

    def _chunk(c, rsegs_outer):
        p = lax.rem(c, NBUF)
        for b in range(NBUF):

            @pl.when(p == b)
            def _(b=b):
                pltpu.make_async_copy(
                    table_hbm.at[pl.ds(0, CHUNK)],
                    h_bufs.at[pl.ds(b * CHUNK, CHUNK)], sems_h[b]).wait()
                pltpu.make_async_copy(
                    table_hbm.at[pl.ds(0, CHUNK)],
                    t_bufs.at[pl.ds(b * CHUNK, CHUNK)], sems_t[b]).wait()

        boff = p * CHUNK

        def _groups(g, rsegs, c=c):
            b0 = boff + g * L
            for ub in range(0, L, 4):
                accs = [None] * 4
                for k in range(N_SEG):
                    for u in range(4):
                        p_ = (h_bufs[b0 + ub + u, pl.ds(k * L, L)]
                              * t_bufs[b0 + ub + u, pl.ds(k * L, L)]
                              * rsegs[k])
                        accs[u] = p_ if k == 0 else accs[u] + p_
                for u in range(4):
                    plsc.store_scatter(tr_v, [tr_idx + ub + u], accs[u])
            sv = tr_v[pl.ds(0, L)]
            for l in range(1, L):
                sv = sv + tr_v[pl.ds(l * TR_STRIDE, L)]
            o_v[pl.ds(c * CHUNK + g * L, L)] = sv
            return rsegs

        rsegs_cur = lax.fori_loop(0, CHUNK // L, _groups, rsegs_outer)

        for b in range(NBUF):

            @pl.when((p == b) & (c + NBUF < N_CHUNKS))
            def _(b=b):
                _issue(c + NBUF, b)

        return rsegs_cur

    lax.fori_loop(0, N_CHUNKS, _chunk, rsegs0)

    pltpu.sync_copy(o_v, out_hbm.at[pl.ds(base, B_PER_W)])


def kernel(head_indices, tail_indices, node_embedding, relation_vector):
    return _distmult_sc(head_indices, tail_indices, node_embedding,
                        relation_vector)
